# Initial kernel scaffold; baseline (speedup 1.0000x reference)
import jax
import jax.numpy as jnp
from jax.experimental import pallas as pl


def kernel(n_feat, edge_index, e_feat, W1, al1, ar1, b1, W2, al2, ar2, b2):
    raise NotImplementedError("write your pallas kernel here")


# speedup vs baseline: 67.7706x; 67.7706x over previous
import functools

import jax
import jax.numpy as jnp
from jax import lax
from jax.experimental import pallas as pl
from jax.experimental.pallas import tpu as pltpu
from jax.experimental.pallas import tpu_sc as plsc

N = 10000
E = 320000
D_IN = 128
H = 8
F = 16
D = H * F
CW = D + 16
NC = 2
NS = 16
NW = NC * NS
EPW = E // NW
K = 80
NCHUNK = EPW // K
NPAD = 10240
RPS = NPAD // NS
RCHUNK = 128
NRC = RPS // RCHUNK


def _leaky(x, slope):
    return jnp.where(x >= 0, x, slope * x)


def _tc_front_body(x_ref, w_ref, alr, arr, hx_ref, er_ref, lmax_ref):
    h = jnp.dot(x_ref[...], w_ref[...], preferred_element_type=jnp.float32)
    el = jnp.dot(h, alr[...], preferred_element_type=jnp.float32)
    er = jnp.dot(h, arr[...], preferred_element_type=jnp.float32)
    hx_ref[:, :D] = h
    hx_ref[:, D:] = el
    er_ref[...] = er
    gm = jnp.max(el, axis=0, keepdims=True) + jnp.max(er, axis=0, keepdims=True)
    lmax_ref[...] = _leaky(gm, 0.2)


def _tc_front(x, W, ALe, ARe):
    return pl.pallas_call(
        _tc_front_body,
        out_shape=(
            jax.ShapeDtypeStruct((N, CW), jnp.float32),
            jax.ShapeDtypeStruct((N, 16), jnp.float32),
            jax.ShapeDtypeStruct((1, 16), jnp.float32),
        ),
    )(x, W, ALe, ARe)


def _tc_mid_body(p_ref, b_ref, r_ref, w_ref, alr, arr, hx_ref, er_ref, lmax_ref):
    acc = p_ref[0, :N] + p_ref[1, :N]
    denf = jnp.dot(acc[:, D:], r_ref[...],
                   preferred_element_type=jnp.float32)
    x = acc[:, :D] / (denf + 1e-9) + b_ref[...]
    x = _leaky(x, 0.01)
    h = jnp.dot(x, w_ref[...], preferred_element_type=jnp.float32)
    el = jnp.dot(h, alr[...], preferred_element_type=jnp.float32)
    er = jnp.dot(h, arr[...], preferred_element_type=jnp.float32)
    hx_ref[:, :D] = h
    hx_ref[:, D:] = el
    er_ref[...] = er
    gm = jnp.max(el, axis=0, keepdims=True) + jnp.max(er, axis=0, keepdims=True)
    lmax_ref[...] = _leaky(gm, 0.2)


def _tc_mid(part, b1, R, W, ALe, ARe):
    return pl.pallas_call(
        _tc_mid_body,
        out_shape=(
            jax.ShapeDtypeStruct((N, CW), jnp.float32),
            jax.ShapeDtypeStruct((N, 16), jnp.float32),
            jax.ShapeDtypeStruct((1, 16), jnp.float32),
        ),
    )(part, b1, R, W, ALe, ARe)


def _tc_final_body(p_ref, b_ref, r_ref, out_ref):
    acc = p_ref[0, :N] + p_ref[1, :N]
    denf = jnp.dot(acc[:, D:], r_ref[...], preferred_element_type=jnp.float32)
    out_ref[...] = acc[:, :D] / (denf + 1e-9) + b_ref[...]


def _tc_final(part, b2, R):
    return pl.pallas_call(
        _tc_final_body,
        out_shape=jax.ShapeDtypeStruct((N, D), jnp.float32),
    )(part, b2, R)


def _bcast_lane(v, h):
    return jnp.take_along_axis(
        v, jnp.full((16,), h, dtype=jnp.int32), axis=0,
        mode="promise_in_bounds")


def _sc_body(hx_hbm, er_hbm, src_hbm, dst_hbm, lmax_hbm, out_hbm,
             srcv, dstv, hrows, errows, lmaxv, stage, acc, sem1, sem2):
    c = lax.axis_index("c")
    s = lax.axis_index("s")
    wid = c * NS + s

    def zrow(i, _):
        for j in range(CW // 16):
            stage[i, pl.ds(j * 16, 16)] = jnp.zeros((16,), jnp.float32)
        return 0
    lax.fori_loop(0, RCHUNK, zrow, 0)

    def zcopy(t, _):
        pltpu.sync_copy(stage, acc.at[pl.ds(s * RPS + t * RCHUNK, RCHUNK)])
        return 0
    lax.fori_loop(0, NRC, zcopy, 0)

    pltpu.sync_copy(lmax_hbm, lmaxv)
    lmax = lmaxv[...]
    plsc.subcore_barrier()

    def chunk(i, _):
        eb = wid * EPW + i * K
        pltpu.sync_copy(src_hbm.at[pl.ds(eb, K)], srcv)
        pltpu.sync_copy(dst_hbm.at[pl.ds(eb, K)], dstv)
        cp1 = pltpu.async_copy(hx_hbm.at[srcv], hrows, sem1)
        cp2 = pltpu.async_copy(er_hbm.at[dstv], errows, sem2)
        cp1.wait()
        cp2.wait()

        def edge(k, _):
            el = hrows[k, pl.ds(D, 16)]
            e = el + errows[k, :]
            e = _leaky(e, 0.2)
            w = jnp.exp(e - lmax)
            hrows[k, pl.ds(D, 16)] = w
            for h in range(H):
                wh = _bcast_lane(w, h)
                hrows[k, pl.ds(h * F, F)] = hrows[k, pl.ds(h * F, F)] * wh
            return 0
        lax.fori_loop(0, K, edge, 0)

        pltpu.sync_copy(hrows, acc.at[dstv], add=True)
        return 0
    lax.fori_loop(0, NCHUNK, chunk, 0)

    plsc.subcore_barrier()

    def drain(t, _):
        rb = s * RPS + t * RCHUNK
        pltpu.sync_copy(acc.at[pl.ds(rb, RCHUNK)], stage)
        pltpu.sync_copy(stage, out_hbm.at[c, pl.ds(rb, RCHUNK)])
        return 0
    lax.fori_loop(0, NRC, drain, 0)


def _sc_aggregate(hx, er, src, dst, lmax):
    mesh = plsc.VectorSubcoreMesh(core_axis_name="c", subcore_axis_name="s")
    f = pl.kernel(
        _sc_body,
        out_type=jax.ShapeDtypeStruct((NC, NPAD, CW), jnp.float32),
        mesh=mesh,
        compiler_params=pltpu.CompilerParams(use_tc_tiling_on_sc=False),
        scratch_types=[
            pltpu.VMEM((K,), jnp.int32),
            pltpu.VMEM((K,), jnp.int32),
            pltpu.VMEM((K, CW), jnp.float32),
            pltpu.VMEM((K, 16), jnp.float32),
            pltpu.VMEM((16,), jnp.float32),
            pltpu.VMEM((RCHUNK, CW), jnp.float32),
            pltpu.VMEM_SHARED((NPAD, CW), jnp.float32),
            pltpu.SemaphoreType.DMA,
            pltpu.SemaphoreType.DMA,
        ],
    )
    return f(hx, er, src, dst, lmax)


def _expand_att(a):
    m = jnp.zeros((D, 16), jnp.float32)
    rows = jnp.arange(D)
    return m.at[rows, rows // F].set(a.reshape(D))


def _expand_rep():
    cols = jnp.arange(D)
    m = jnp.zeros((16, D), jnp.float32)
    return m.at[cols // F, cols].set(1.0)


@jax.jit
def kernel(n_feat, edge_index, e_feat, W1, al1, ar1, b1, W2, al2, ar2, b2):
    del e_feat
    src = edge_index[0].astype(jnp.int32)
    dst = edge_index[1].astype(jnp.int32)
    AL1, AR1 = _expand_att(al1), _expand_att(ar1)
    AL2, AR2 = _expand_att(al2), _expand_att(ar2)
    R = _expand_rep()
    b1r = b1.reshape(1, D)
    b2r = b2.reshape(1, D)

    hx1, er1, lmax1 = _tc_front(n_feat, W1, AL1, AR1)
    part1 = _sc_aggregate(hx1, er1, src, dst, lmax1.reshape(16))
    hx2, er2, lmax2 = _tc_mid(part1, b1r, R, W2, AL2, AR2)
    part2 = _sc_aggregate(hx2, er2, src, dst, lmax2.reshape(16))
    return _tc_final(part2, b2r, R)


# speedup vs baseline: 91.7355x; 1.3536x over previous
import functools

import jax
import jax.numpy as jnp
from jax import lax
from jax.experimental import pallas as pl
from jax.experimental.pallas import tpu as pltpu
from jax.experimental.pallas import tpu_sc as plsc

N = 10000
E = 320000
D_IN = 128
H = 8
F = 16
D = H * F
CW = D + 16
NC = 2
NS = 16
NW = NC * NS
EPW = E // NW
K = 80
NCHUNK = EPW // K
NPAD = 10240
RPS = NPAD // NS
RCHUNK = 32
NRC = RPS // RCHUNK


def _leaky(x, slope):
    return jnp.where(x >= 0, x, slope * x)


def _tc_front_body(x_ref, w_ref, alr, arr, hx_ref, er_ref, lmax_ref):
    h = jnp.dot(x_ref[...], w_ref[...], preferred_element_type=jnp.float32)
    el = jnp.dot(h, alr[...], preferred_element_type=jnp.float32)
    er = jnp.dot(h, arr[...], preferred_element_type=jnp.float32)
    hx_ref[:, :D] = h
    hx_ref[:, D:] = el
    er_ref[...] = er
    gm = jnp.max(el, axis=0, keepdims=True) + jnp.max(er, axis=0, keepdims=True)
    lmax_ref[...] = _leaky(gm, 0.2)


def _tc_front(x, W, ALe, ARe):
    return pl.pallas_call(
        _tc_front_body,
        out_shape=(
            jax.ShapeDtypeStruct((N, CW), jnp.float32),
            jax.ShapeDtypeStruct((N, 16), jnp.float32),
            jax.ShapeDtypeStruct((1, 16), jnp.float32),
        ),
    )(x, W, ALe, ARe)


def _tc_mid_body(p_ref, b_ref, r_ref, w_ref, alr, arr, hx_ref, er_ref, lmax_ref):
    acc = p_ref[0, :N] + p_ref[1, :N]
    denf = jnp.dot(acc[:, D:], r_ref[...],
                   preferred_element_type=jnp.float32)
    x = acc[:, :D] / (denf + 1e-9) + b_ref[...]
    x = _leaky(x, 0.01)
    h = jnp.dot(x, w_ref[...], preferred_element_type=jnp.float32)
    el = jnp.dot(h, alr[...], preferred_element_type=jnp.float32)
    er = jnp.dot(h, arr[...], preferred_element_type=jnp.float32)
    hx_ref[:, :D] = h
    hx_ref[:, D:] = el
    er_ref[...] = er
    gm = jnp.max(el, axis=0, keepdims=True) + jnp.max(er, axis=0, keepdims=True)
    lmax_ref[...] = _leaky(gm, 0.2)


def _tc_mid(part, b1, R, W, ALe, ARe):
    return pl.pallas_call(
        _tc_mid_body,
        out_shape=(
            jax.ShapeDtypeStruct((N, CW), jnp.float32),
            jax.ShapeDtypeStruct((N, 16), jnp.float32),
            jax.ShapeDtypeStruct((1, 16), jnp.float32),
        ),
    )(part, b1, R, W, ALe, ARe)


def _tc_final_body(p_ref, b_ref, r_ref, out_ref):
    acc = p_ref[0, :N] + p_ref[1, :N]
    denf = jnp.dot(acc[:, D:], r_ref[...], preferred_element_type=jnp.float32)
    out_ref[...] = acc[:, :D] / (denf + 1e-9) + b_ref[...]


def _tc_final(part, b2, R):
    return pl.pallas_call(
        _tc_final_body,
        out_shape=jax.ShapeDtypeStruct((N, D), jnp.float32),
    )(part, b2, R)


def _bcast_lane(v, h):
    return jnp.take_along_axis(
        v, jnp.full((16,), h, dtype=jnp.int32), axis=0,
        mode="promise_in_bounds")


def _sc_body(hx_hbm, er_hbm, src_hbm, dst_hbm, lmax_hbm, out_hbm,
             srcv0, dstv0, srcv1, dstv1,
             hrows0, errows0, hrows1, errows1,
             lmaxv, stage, acc,
             semi0, semi1, semg0, semg1, semz):
    c = lax.axis_index("c")
    s = lax.axis_index("s")
    wid = c * NS + s
    eb0 = wid * EPW

    def zrow(i, _):
        for j in range(CW // 16):
            stage[i, pl.ds(j * 16, 16)] = jnp.zeros((16,), jnp.float32)
        return 0
    lax.fori_loop(0, RCHUNK, zrow, 0)
    for t in range(NRC):
        pltpu.async_copy(stage, acc.at[pl.ds(s * RPS + t * RCHUNK, RCHUNK)],
                         semz)
    for t in range(NRC):
        pltpu.make_async_copy(
            stage, acc.at[pl.ds(s * RPS + t * RCHUNK, RCHUNK)], semz).wait()

    pltpu.sync_copy(lmax_hbm, lmaxv)
    lmax = lmaxv[...]
    plsc.subcore_barrier()

    def idx_copy(i, sv, dv, sem):
        eb = eb0 + i * K
        pltpu.async_copy(src_hbm.at[pl.ds(eb, K)], sv, sem)
        pltpu.async_copy(dst_hbm.at[pl.ds(eb, K)], dv, sem)

    def idx_wait(sv, dv, sem):
        pltpu.make_async_copy(src_hbm.at[pl.ds(0, K)], sv, sem).wait()
        pltpu.make_async_copy(dst_hbm.at[pl.ds(0, K)], dv, sem).wait()

    def gather_start(sv, dv, hb, erb, sem):
        pltpu.async_copy(hx_hbm.at[sv], hb, sem)
        pltpu.async_copy(er_hbm.at[dv], erb, sem)

    def gather_wait(sv, dv, hb, erb, sem):
        pltpu.make_async_copy(hx_hbm.at[sv], hb, sem).wait()
        pltpu.make_async_copy(er_hbm.at[dv], erb, sem).wait()

    def compute_scatter(hb, erb, dv):
        def edge(k, _):
            el = hb[k, pl.ds(D, 16)]
            e = el + erb[k, :]
            e = _leaky(e, 0.2)
            w = jnp.exp(e - lmax)
            hb[k, pl.ds(D, 16)] = w
            for h in range(H):
                wh = _bcast_lane(w, h)
                hb[k, pl.ds(h * F, F)] = hb[k, pl.ds(h * F, F)] * wh
            return 0
        lax.fori_loop(0, K, edge, 0, unroll=4)
        pltpu.sync_copy(hb, acc.at[dv], add=True)

    idx_copy(0, srcv0, dstv0, semi0)
    idx_wait(srcv0, dstv0, semi0)
    gather_start(srcv0, dstv0, hrows0, errows0, semg0)
    idx_copy(1, srcv1, dstv1, semi1)

    def pair(j, _):
        idx_wait(srcv1, dstv1, semi1)
        gather_start(srcv1, dstv1, hrows1, errows1, semg1)
        gather_wait(srcv0, dstv0, hrows0, errows0, semg0)
        compute_scatter(hrows0, errows0, dstv0)
        idx_copy(2 * j + 2, srcv0, dstv0, semi0)
        gather_wait(srcv1, dstv1, hrows1, errows1, semg1)
        compute_scatter(hrows1, errows1, dstv1)
        idx_copy(jnp.minimum(2 * j + 3, NCHUNK - 1), srcv1, dstv1, semi1)
        idx_wait(srcv0, dstv0, semi0)
        gather_start(srcv0, dstv0, hrows0, errows0, semg0)
        return 0
    lax.fori_loop(0, (NCHUNK - 1) // 2, pair, 0)

    gather_wait(srcv0, dstv0, hrows0, errows0, semg0)
    compute_scatter(hrows0, errows0, dstv0)
    idx_wait(srcv1, dstv1, semi1)

    plsc.subcore_barrier()

    pltpu.sync_copy(acc.at[pl.ds(s * RPS, RPS)],
                    out_hbm.at[c, pl.ds(s * RPS, RPS)])


def _sc_aggregate(hx, er, src, dst, lmax):
    mesh = plsc.VectorSubcoreMesh(core_axis_name="c", subcore_axis_name="s")
    f = pl.kernel(
        _sc_body,
        out_type=jax.ShapeDtypeStruct((NC, NPAD, CW), jnp.float32),
        mesh=mesh,
        compiler_params=pltpu.CompilerParams(use_tc_tiling_on_sc=False),
        scratch_types=[
            pltpu.VMEM((K,), jnp.int32),
            pltpu.VMEM((K,), jnp.int32),
            pltpu.VMEM((K,), jnp.int32),
            pltpu.VMEM((K,), jnp.int32),
            pltpu.VMEM((K, CW), jnp.float32),
            pltpu.VMEM((K, 16), jnp.float32),
            pltpu.VMEM((K, CW), jnp.float32),
            pltpu.VMEM((K, 16), jnp.float32),
            pltpu.VMEM((16,), jnp.float32),
            pltpu.VMEM((RCHUNK, CW), jnp.float32),
            pltpu.VMEM_SHARED((NPAD, CW), jnp.float32),
            pltpu.SemaphoreType.DMA,
            pltpu.SemaphoreType.DMA,
            pltpu.SemaphoreType.DMA,
            pltpu.SemaphoreType.DMA,
            pltpu.SemaphoreType.DMA,
        ],
    )
    return f(hx, er, src, dst, lmax)


def _expand_att(a):
    m = jnp.zeros((D, 16), jnp.float32)
    rows = jnp.arange(D)
    return m.at[rows, rows // F].set(a.reshape(D))


def _expand_rep():
    cols = jnp.arange(D)
    m = jnp.zeros((16, D), jnp.float32)
    return m.at[cols // F, cols].set(1.0)


@jax.jit
def kernel(n_feat, edge_index, e_feat, W1, al1, ar1, b1, W2, al2, ar2, b2):
    del e_feat
    src = edge_index[0].astype(jnp.int32)
    dst = edge_index[1].astype(jnp.int32)
    AL1, AR1 = _expand_att(al1), _expand_att(ar1)
    AL2, AR2 = _expand_att(al2), _expand_att(ar2)
    R = _expand_rep()
    b1r = b1.reshape(1, D)
    b2r = b2.reshape(1, D)

    hx1, er1, lmax1 = _tc_front(n_feat, W1, AL1, AR1)
    part1 = _sc_aggregate(hx1, er1, src, dst, lmax1.reshape(16))
    hx2, er2, lmax2 = _tc_mid(part1, b1r, R, W2, AL2, AR2)
    part2 = _sc_aggregate(hx2, er2, src, dst, lmax2.reshape(16))
    return _tc_final(part2, b2r, R)


# speedup vs baseline: 161.4185x; 1.7596x over previous
import functools

import jax
import jax.numpy as jnp
from jax import lax
from jax.experimental import pallas as pl
from jax.experimental.pallas import tpu as pltpu
from jax.experimental.pallas import tpu_sc as plsc

N = 10000
E = 320000
D_IN = 128
H = 8
F = 16
D = H * F
CW = D + 16
NC = 2
NS = 16
NW = NC * NS
EPW = E // NW
K = 80
NCHUNK = EPW // K
NPAD = 10240
RPS = NPAD // NS
RCHUNK = 32
NRC = RPS // RCHUNK


def _leaky(x, slope):
    return jnp.where(x >= 0, x, slope * x)


def _tc_front_body(x_ref, w_ref, alr, arr, hx_ref, er_ref, lmax_ref):
    h = jnp.dot(x_ref[...], w_ref[...], preferred_element_type=jnp.float32)
    el = jnp.dot(h, alr[...], preferred_element_type=jnp.float32)
    er = jnp.dot(h, arr[...], preferred_element_type=jnp.float32)
    hx_ref[:, :D] = h
    hx_ref[:, D:] = el
    er_ref[...] = er
    gm = jnp.max(el, axis=0, keepdims=True) + jnp.max(er, axis=0, keepdims=True)
    lmax_ref[...] = _leaky(gm, 0.2)


def _tc_front(x, W, ALe, ARe):
    return pl.pallas_call(
        _tc_front_body,
        out_shape=(
            jax.ShapeDtypeStruct((N, CW), jnp.float32),
            jax.ShapeDtypeStruct((N, 16), jnp.float32),
            jax.ShapeDtypeStruct((1, 16), jnp.float32),
        ),
    )(x, W, ALe, ARe)


def _tc_mid_body(p_ref, b_ref, r_ref, w_ref, alr, arr, hx_ref, er_ref, lmax_ref):
    acc = p_ref[0, :N] + p_ref[1, :N]
    denf = jnp.dot(acc[:, D:], r_ref[...],
                   preferred_element_type=jnp.float32)
    x = acc[:, :D] / (denf + 1e-9) + b_ref[...]
    x = _leaky(x, 0.01)
    h = jnp.dot(x, w_ref[...], preferred_element_type=jnp.float32)
    el = jnp.dot(h, alr[...], preferred_element_type=jnp.float32)
    er = jnp.dot(h, arr[...], preferred_element_type=jnp.float32)
    hx_ref[:, :D] = h
    hx_ref[:, D:] = el
    er_ref[...] = er
    gm = jnp.max(el, axis=0, keepdims=True) + jnp.max(er, axis=0, keepdims=True)
    lmax_ref[...] = _leaky(gm, 0.2)


def _tc_mid(part, b1, R, W, ALe, ARe):
    return pl.pallas_call(
        _tc_mid_body,
        out_shape=(
            jax.ShapeDtypeStruct((N, CW), jnp.float32),
            jax.ShapeDtypeStruct((N, 16), jnp.float32),
            jax.ShapeDtypeStruct((1, 16), jnp.float32),
        ),
    )(part, b1, R, W, ALe, ARe)


def _tc_final_body(p_ref, b_ref, r_ref, out_ref):
    acc = p_ref[0, :N] + p_ref[1, :N]
    denf = jnp.dot(acc[:, D:], r_ref[...], preferred_element_type=jnp.float32)
    out_ref[...] = acc[:, :D] / (denf + 1e-9) + b_ref[...]


def _tc_final(part, b2, R):
    return pl.pallas_call(
        _tc_final_body,
        out_shape=jax.ShapeDtypeStruct((N, D), jnp.float32),
    )(part, b2, R)


def _bcast_lane(v, h):
    return jnp.take_along_axis(
        v, jnp.full((16,), h, dtype=jnp.int32), axis=0,
        mode="promise_in_bounds")


def _sc_body(hx_hbm, er_hbm, src_hbm, dst_hbm, lmax_hbm, out_hbm,
             srcv0, dstv0, srcv1, dstv1, dstsc0, dstsc1,
             hrows0, errows0, hrows1, errows1,
             lmaxv, stage, acc,
             semi0, semi1, semg0, semg1, semsc0, semsc1, semz):
    c = lax.axis_index("c")
    s = lax.axis_index("s")
    wid = c * NS + s
    eb0 = wid * EPW

    def zrow(i, _):
        for j in range(CW // 16):
            stage[i, pl.ds(j * 16, 16)] = jnp.zeros((16,), jnp.float32)
        return 0
    lax.fori_loop(0, RCHUNK, zrow, 0)
    for t in range(NRC):
        pltpu.async_copy(stage, acc.at[pl.ds(s * RPS + t * RCHUNK, RCHUNK)],
                         semz)
    for t in range(NRC):
        pltpu.make_async_copy(
            stage, acc.at[pl.ds(s * RPS + t * RCHUNK, RCHUNK)], semz).wait()

    pltpu.sync_copy(lmax_hbm, lmaxv)
    lmax = lmaxv[...]
    plsc.subcore_barrier()

    def idx_copy(i, sv, dv, sem):
        eb = eb0 + i * K
        pltpu.async_copy(src_hbm.at[pl.ds(eb, K)], sv, sem)
        pltpu.async_copy(dst_hbm.at[pl.ds(eb, K)], dv, sem)

    def idx_wait(sv, dv, sem):
        pltpu.make_async_copy(src_hbm.at[pl.ds(0, K)], sv, sem).wait()
        pltpu.make_async_copy(dst_hbm.at[pl.ds(0, K)], dv, sem).wait()

    def gather_start(sv, dv, hb, erb, sem):
        pltpu.async_copy(hx_hbm.at[sv], hb, sem)
        pltpu.async_copy(er_hbm.at[dv], erb, sem)

    def gather_wait(sv, dv, hb, erb, sem):
        pltpu.make_async_copy(hx_hbm.at[sv], hb, sem).wait()
        pltpu.make_async_copy(er_hbm.at[dv], erb, sem).wait()

    def compute(hb, erb):
        @plsc.parallel_loop(0, K, unroll=4)
        def edge(k):
            el = hb[k, pl.ds(D, 16)]
            e = el + erb[k, :]
            e = _leaky(e, 0.2)
            w = jnp.exp(e - lmax)
            hb[k, pl.ds(D, 16)] = w
            for h in range(H):
                wh = _bcast_lane(w, h)
                hb[k, pl.ds(h * F, F)] = hb[k, pl.ds(h * F, F)] * wh

    def dst_snapshot(dv, dsc):
        for j in range(K // 16):
            dsc[pl.ds(j * 16, 16)] = dv[pl.ds(j * 16, 16)]

    def scatter_start(hb, dsc, sem):
        pltpu.async_copy(hb, acc.at[dsc], sem, add=True)

    def scatter_wait(hb, dsc, sem):
        pltpu.make_async_copy(hb, acc.at[dsc], sem).wait()

    idx_copy(0, srcv0, dstv0, semi0)
    idx_wait(srcv0, dstv0, semi0)
    gather_start(srcv0, dstv0, hrows0, errows0, semg0)
    idx_copy(1, srcv1, dstv1, semi1)

    def pair(j, _):
        @pl.when(j > 0)
        def _():
            scatter_wait(hrows1, dstsc1, semsc1)
        idx_wait(srcv1, dstv1, semi1)
        gather_start(srcv1, dstv1, hrows1, errows1, semg1)
        gather_wait(srcv0, dstv0, hrows0, errows0, semg0)
        compute(hrows0, errows0)
        dst_snapshot(dstv0, dstsc0)
        scatter_start(hrows0, dstsc0, semsc0)
        idx_copy(2 * j + 2, srcv0, dstv0, semi0)
        gather_wait(srcv1, dstv1, hrows1, errows1, semg1)
        compute(hrows1, errows1)
        dst_snapshot(dstv1, dstsc1)
        scatter_start(hrows1, dstsc1, semsc1)
        idx_copy(jnp.minimum(2 * j + 3, NCHUNK - 1), srcv1, dstv1, semi1)
        scatter_wait(hrows0, dstsc0, semsc0)
        idx_wait(srcv0, dstv0, semi0)
        gather_start(srcv0, dstv0, hrows0, errows0, semg0)
        return 0
    lax.fori_loop(0, (NCHUNK - 1) // 2, pair, 0)

    scatter_wait(hrows1, dstsc1, semsc1)
    gather_wait(srcv0, dstv0, hrows0, errows0, semg0)
    compute(hrows0, errows0)
    pltpu.sync_copy(hrows0, acc.at[dstv0], add=True)
    idx_wait(srcv1, dstv1, semi1)

    plsc.subcore_barrier()

    pltpu.sync_copy(acc.at[pl.ds(s * RPS, RPS)],
                    out_hbm.at[c, pl.ds(s * RPS, RPS)])


def _sc_aggregate(hx, er, src, dst, lmax):
    mesh = plsc.VectorSubcoreMesh(core_axis_name="c", subcore_axis_name="s")
    f = pl.kernel(
        _sc_body,
        out_type=jax.ShapeDtypeStruct((NC, NPAD, CW), jnp.float32),
        mesh=mesh,
        compiler_params=pltpu.CompilerParams(use_tc_tiling_on_sc=False),
        scratch_types=[
            pltpu.VMEM((K,), jnp.int32),
            pltpu.VMEM((K,), jnp.int32),
            pltpu.VMEM((K,), jnp.int32),
            pltpu.VMEM((K,), jnp.int32),
            pltpu.VMEM((K,), jnp.int32),
            pltpu.VMEM((K,), jnp.int32),
            pltpu.VMEM((K, CW), jnp.float32),
            pltpu.VMEM((K, 16), jnp.float32),
            pltpu.VMEM((K, CW), jnp.float32),
            pltpu.VMEM((K, 16), jnp.float32),
            pltpu.VMEM((16,), jnp.float32),
            pltpu.VMEM((RCHUNK, CW), jnp.float32),
            pltpu.VMEM_SHARED((NPAD, CW), jnp.float32),
            pltpu.SemaphoreType.DMA,
            pltpu.SemaphoreType.DMA,
            pltpu.SemaphoreType.DMA,
            pltpu.SemaphoreType.DMA,
            pltpu.SemaphoreType.DMA,
            pltpu.SemaphoreType.DMA,
            pltpu.SemaphoreType.DMA,
        ],
    )
    return f(hx, er, src, dst, lmax)


def _expand_att(a):
    m = jnp.zeros((D, 16), jnp.float32)
    rows = jnp.arange(D)
    return m.at[rows, rows // F].set(a.reshape(D))


def _expand_rep():
    cols = jnp.arange(D)
    m = jnp.zeros((16, D), jnp.float32)
    return m.at[cols // F, cols].set(1.0)


@jax.jit
def kernel(n_feat, edge_index, e_feat, W1, al1, ar1, b1, W2, al2, ar2, b2):
    del e_feat
    src = edge_index[0].astype(jnp.int32)
    dst = edge_index[1].astype(jnp.int32)
    AL1, AR1 = _expand_att(al1), _expand_att(ar1)
    AL2, AR2 = _expand_att(al2), _expand_att(ar2)
    R = _expand_rep()
    b1r = b1.reshape(1, D)
    b2r = b2.reshape(1, D)

    hx1, er1, lmax1 = _tc_front(n_feat, W1, AL1, AR1)
    part1 = _sc_aggregate(hx1, er1, src, dst, lmax1.reshape(16))
    hx2, er2, lmax2 = _tc_mid(part1, b1r, R, W2, AL2, AR2)
    part2 = _sc_aggregate(hx2, er2, src, dst, lmax2.reshape(16))
    return _tc_final(part2, b2r, R)
